# baseline (speedup 1.0000x reference)
import functools

import jax
import jax.numpy as jnp
from jax import lax
from jax.experimental import pallas as pl
from jax.experimental.pallas import tpu as pltpu
from jax.experimental.pallas import tpu_sc as plsc

_NC, _NS = 2, 16
_NW = _NC * _NS

_B, _H, _D, _V = 16384, 50, 64, 100000
_BPW = _B // _NW
_IPW = _BPW * _H
_UNITS = 2 * _H


def _transform_table(table, W, b2d):
    V, E = table.shape
    O = W.shape[0]
    BLK = 4000

    def body(t_ref, w_ref, b_ref, o_ref):
        x = t_ref[...]
        row = lax.broadcasted_iota(jnp.int32, x.shape, 0)
        x = jnp.where((row == 0) & (pl.program_id(0) == 0), 0.0, x)
        y = lax.dot_general(x, w_ref[...], (((1,), (1,)), ((), ())),
                            preferred_element_type=jnp.float32)
        o_ref[...] = jnp.maximum(y + b_ref[...], 0.0)

    return pl.pallas_call(
        body,
        grid=(V // BLK,),
        in_specs=[
            pl.BlockSpec((BLK, E), lambda i: (i, 0)),
            pl.BlockSpec((O, E), lambda i: (0, 0)),
            pl.BlockSpec((1, O), lambda i: (0, 0)),
        ],
        out_specs=pl.BlockSpec((BLK, O), lambda i: (i, 0)),
        out_shape=jax.ShapeDtypeStruct((V, O), jnp.float32),
    )(table, W, b2d)


def _make_tgather():
    mesh = plsc.VectorSubcoreMesh(core_axis_name="c", subcore_axis_name="s")

    @functools.partial(
        pl.kernel,
        mesh=mesh,
        out_type=jax.ShapeDtypeStruct((_H, _D // 8, _B // 128, 8, 128),
                                      jnp.float32),
        compiler_params=pltpu.CompilerParams(use_tc_tiling_on_sc=False,
                                             needs_layout_passes=False,
                                             disable_bounds_checks=True),
        scratch_types=[
            pltpu.VMEM((_IPW,), jnp.int32),
            pltpu.VMEM((_IPW,), jnp.int32),
            pltpu.VMEM((256, _D), jnp.float32),
            pltpu.VMEM((256, _D), jnp.float32),
            pltpu.VMEM((8, 2, 8, 128), jnp.float32),
            pltpu.VMEM((8, 2, 8, 128), jnp.float32),
            pltpu.SemaphoreType.DMA,
            pltpu.SemaphoreType.DMA,
            pltpu.SemaphoreType.DMA,
            pltpu.SemaphoreType.DMA,
        ],
    )
    def tgather(t2_hbm, idx_hbm, out_hbm, idx_raw, idx_t,
                rows0, rows1, trans0, trans1, gs0, gs1, ws0, ws1):
        wid = lax.axis_index("s") * _NC + lax.axis_index("c")
        base = wid * _IPW
        lane = jnp.arange(16, dtype=jnp.int32)

        pltpu.sync_copy(idx_hbm.at[pl.ds(base, _IPW)], idx_raw)

        def blc_body(blc, _):
            bl0 = blc * 16
            iv0 = bl0 * _H + lane * _H

            def h_body(h, iv):
                vals = plsc.load_gather(idx_raw, [iv])
                idx_t[pl.ds(h * _BPW + bl0, 16)] = vals
                return iv + 1

            lax.fori_loop(0, _H, h_body, iv0)
            return _

        lax.fori_loop(0, _BPW // 16, blc_body, 0)

        rows = (rows0, rows1)
        trans = (trans0, trans1)
        gsem = (gs0, gs1)
        wsem = (ws0, ws1)

        def fire_gather(u, slot):
            off = u * 256
            for s in range(2):
                pltpu.async_copy(
                    t2_hbm.at[idx_t.at[pl.ds(off + s * 128, 128)]],
                    rows[slot].at[pl.ds(s * 128, 128)],
                    gsem[slot],
                )

        def wait_gather(slot):
            pltpu.make_async_copy(
                t2_hbm.at[pl.ds(0, 256)], rows[slot], gsem[slot]).wait()

        rvecs = [c * 128 + bic * 16 + lane
                 for c in range(2) for bic in range(8)]
        zerov = jnp.zeros((16,), jnp.int32)

        def transpose(slot):
            r = rows[slot]
            t = trans[slot]

            def di_body(di, carry):
                cv0 = zerov + di
                for c in range(2):
                    for bic in range(8):
                        rv = rvecs[c * 8 + bic]
                        for dt in range(8):
                            vals = plsc.load_gather(r, [rv, cv0 + dt * 8])
                            t[dt, c, di, pl.ds(bic * 16, 16)] = vals
                return carry

            lax.fori_loop(0, 8, di_body, 0)

        def fire_writes(u, slot):
            h = u // 2
            half = u % 2
            bt0 = wid * 4 + half * 2
            for dt in range(8):
                pltpu.async_copy(
                    trans[slot].at[dt],
                    out_hbm.at[h, dt, pl.ds(bt0, 2)],
                    wsem[slot],
                )

        def wait_writes(u, slot):
            h = u // 2
            half = u % 2
            bt0 = wid * 4 + half * 2
            for dt in range(8):
                pltpu.make_async_copy(
                    trans[slot].at[dt],
                    out_hbm.at[h, dt, pl.ds(bt0, 2)],
                    wsem[slot],
                ).wait()

        fire_gather(0, 0)

        def unit_pair(i, carry):
            u0 = i * 2
            fire_gather(u0 + 1, 1)
            wait_gather(0)

            @pl.when(i > 0)
            def _w0():
                wait_writes(u0 - 2, 0)

            transpose(0)
            fire_writes(u0, 0)

            @pl.when(i + 1 < _UNITS // 2)
            def _g0():
                fire_gather(u0 + 2, 0)

            wait_gather(1)

            @pl.when(i > 0)
            def _w1():
                wait_writes(u0 - 1, 1)

            transpose(1)
            fire_writes(u0 + 1, 1)
            return carry

        lax.fori_loop(0, _UNITS // 2, unit_pair, 0)
        wait_writes(_UNITS - 2, 0)
        wait_writes(_UNITS - 1, 1)

    return tgather


def kernel(category, table, W, b):
    B, H = category.shape
    D = W.shape[0]
    assert (B, H, D) == (_B, _H, _D)
    t2 = _transform_table(table, W, b.reshape(1, -1))
    idx = category.reshape(-1).astype(jnp.int32)
    out5 = _make_tgather()(t2, idx)
    return out5.transpose(2, 4, 0, 1, 3).reshape(B, H, D)


# speedup vs baseline: 4.3182x; 4.3182x over previous
import functools

import jax
import jax.numpy as jnp
from jax import lax
from jax.experimental import pallas as pl
from jax.experimental.pallas import tpu as pltpu
from jax.experimental.pallas import tpu_sc as plsc

_NC, _NS = 2, 16
_NW = _NC * _NS

_B, _H, _D, _V = 16384, 50, 64, 100000
_BPW = _B // _NW
_IPW = _BPW * _H
_UNITS = 2 * _H


def _transform_table(table, W, b2d):
    V, E = table.shape
    O = W.shape[0]
    BLK = 4000

    def body(t_ref, w_ref, b_ref, o_ref):
        x = t_ref[...]
        row = lax.broadcasted_iota(jnp.int32, x.shape, 0)
        x = jnp.where((row == 0) & (pl.program_id(0) == 0), 0.0, x)
        y = lax.dot_general(x, w_ref[...], (((1,), (1,)), ((), ())),
                            preferred_element_type=jnp.float32)
        o_ref[...] = jnp.maximum(y + b_ref[...], 0.0)

    return pl.pallas_call(
        body,
        grid=(V // BLK,),
        in_specs=[
            pl.BlockSpec((BLK, E), lambda i: (i, 0)),
            pl.BlockSpec((O, E), lambda i: (0, 0)),
            pl.BlockSpec((1, O), lambda i: (0, 0)),
        ],
        out_specs=pl.BlockSpec((BLK, O), lambda i: (i, 0)),
        out_shape=jax.ShapeDtypeStruct((V, O), jnp.float32),
    )(table, W, b2d)


def _make_tgather():
    mesh = plsc.VectorSubcoreMesh(core_axis_name="c", subcore_axis_name="s")

    @functools.partial(
        pl.kernel,
        mesh=mesh,
        out_type=jax.ShapeDtypeStruct((_H, _D // 8, _B // 128, 8, 128),
                                      jnp.float32),
        compiler_params=pltpu.CompilerParams(use_tc_tiling_on_sc=False,
                                             needs_layout_passes=False,
                                             disable_bounds_checks=True),
        scratch_types=[
            pltpu.VMEM((_IPW,), jnp.int32),
            pltpu.VMEM((_IPW,), jnp.int32),
            pltpu.VMEM((256, _D), jnp.float32),
            pltpu.VMEM((256, _D), jnp.float32),
            pltpu.VMEM((8, 2, 9, 133), jnp.float32),
            pltpu.VMEM((8, 2, 9, 133), jnp.float32),
            pltpu.SemaphoreType.DMA,
            pltpu.SemaphoreType.DMA,
            pltpu.SemaphoreType.DMA,
            pltpu.SemaphoreType.DMA,
        ],
    )
    def tgather(t2_hbm, idx_hbm, out_hbm, idx_raw, idx_t,
                rows0, rows1, trans0, trans1, gs0, gs1, ws0, ws1):
        wid = lax.axis_index("s") * _NC + lax.axis_index("c")
        base = wid * _IPW
        lane = jnp.arange(16, dtype=jnp.int32)

        pltpu.sync_copy(idx_hbm.at[pl.ds(base, _IPW)], idx_raw)

        def blc_body(blc, _):
            bl0 = blc * 16
            iv0 = bl0 * _H + lane * _H

            def h_body(h, iv):
                vals = plsc.load_gather(idx_raw, [iv])
                idx_t[pl.ds(h * _BPW + bl0, 16)] = vals
                return iv + 1

            lax.fori_loop(0, _H, h_body, iv0)
            return _

        lax.fori_loop(0, _BPW // 16, blc_body, 0)

        rows = (rows0, rows1)
        trans = (trans0, trans1)
        gsem = (gs0, gs1)
        wsem = (ws0, ws1)

        def fire_gather(u, slot):
            off = u * 256
            for s in range(2):
                pltpu.async_copy(
                    t2_hbm.at[idx_t.at[pl.ds(off + s * 128, 128)]],
                    rows[slot].at[pl.ds(s * 128, 128)],
                    gsem[slot],
                )

        def wait_gather(slot):
            pltpu.make_async_copy(
                t2_hbm.at[pl.ds(0, 256)], rows[slot], gsem[slot]).wait()

        dt_vecs = [2 * dc + lane // 8 for dc in range(4)]
        cv_vecs = [jnp.full((16,), dc * 16, jnp.int32) + lane for dc in range(4)]
        di_vec = lane % 8
        zerov = jnp.zeros((16,), jnp.int32)

        def transpose(slot):
            r = rows[slot]
            t = trans[slot]

            def c_half(c_local):
                c_vec = jnp.full((16,), c_local, jnp.int32)

                @plsc.parallel_loop(0, 128, step=1, unroll=8)
                def _bl_body(i):
                    bi_vec = zerov + i
                    r_vec = bi_vec + (c_local * 128)
                    for dc in range(4):
                        vals = plsc.load_gather(r, [r_vec, cv_vecs[dc]])
                        plsc.store_scatter(
                            t, [dt_vecs[dc], c_vec, di_vec, bi_vec], vals)

            c_half(0)
            c_half(1)

        def fire_writes(u, slot):
            h = u // 2
            half = u % 2
            bt0 = wid * 4 + half * 2
            for dt in range(8):
                pltpu.async_copy(
                    trans[slot].at[dt, :, pl.ds(0, 8), pl.ds(0, 128)],
                    out_hbm.at[h, dt, pl.ds(bt0, 2)],
                    wsem[slot],
                )

        def wait_writes(u, slot):
            h = u // 2
            half = u % 2
            bt0 = wid * 4 + half * 2
            for dt in range(8):
                pltpu.make_async_copy(
                    trans[slot].at[dt, :, pl.ds(0, 8), pl.ds(0, 128)],
                    out_hbm.at[h, dt, pl.ds(bt0, 2)],
                    wsem[slot],
                ).wait()

        fire_gather(0, 0)

        def unit_pair(i, carry):
            u0 = i * 2
            fire_gather(u0 + 1, 1)
            wait_gather(0)

            @pl.when(i > 0)
            def _w0():
                wait_writes(u0 - 2, 0)

            transpose(0)
            fire_writes(u0, 0)

            @pl.when(i + 1 < _UNITS // 2)
            def _g0():
                fire_gather(u0 + 2, 0)

            wait_gather(1)

            @pl.when(i > 0)
            def _w1():
                wait_writes(u0 - 1, 1)

            transpose(1)
            fire_writes(u0 + 1, 1)
            return carry

        lax.fori_loop(0, _UNITS // 2, unit_pair, 0)
        wait_writes(_UNITS - 2, 0)
        wait_writes(_UNITS - 1, 1)

    return tgather


def kernel(category, table, W, b):
    B, H = category.shape
    D = W.shape[0]
    assert (B, H, D) == (_B, _H, _D)
    t2 = _transform_table(table, W, b.reshape(1, -1))
    idx = category.reshape(-1).astype(jnp.int32)
    out5 = _make_tgather()(t2, idx)
    return out5.transpose(2, 4, 0, 1, 3).reshape(B, H, D)
